# baseline (speedup 1.0000x reference)
import jax
import jax.numpy as jnp
from jax.experimental import pallas as pl
from jax.experimental.pallas import tpu as pltpu

_B, _P, _K = 128, 128, 16
_BP = _B * _P
_NE = _BP * _K
_EPS = 1e-5


def _dot(a, b):
    return jax.lax.dot_general(
        a.astype(jnp.bfloat16), b.astype(jnp.bfloat16),
        (((1,), (0,)), ((), ())), preferred_element_type=jnp.float32)


def _rowstats(x):
    m = jnp.mean(x, axis=1, keepdims=True)
    v = jnp.mean(x * x, axis=1, keepdims=True) - m * m
    return m, v


_G = 8


def _knn_body(pts_ref, ptsT_ref, featF_ref, maskF_ref, bng_ref, bnb_ref,
              idx_ref, x0_ref):
    iq = jax.lax.broadcasted_iota(jnp.int32, (1, _P, 1), 1).astype(jnp.float32)

    def grp(jc, c):
        pj = pts_ref[pl.ds(jc * _G, _G)]
        ptj = ptsT_ref[pl.ds(jc * _G, _G)]
        px, py = pj[:, 0:1, :], pj[:, 1:2, :]
        pxc, pyc = ptj[:, :, 0:1], ptj[:, :, 1:2]
        pair = (2.0 * (pxc * px + pyc * py)
                - (pxc * pxc + pyc * pyc) - (px * px + py * py))
        rows = []
        for t in range(_K + 1):
            m = jnp.max(pair, axis=1, keepdims=True)
            cand = jnp.where(pair == m, jnp.broadcast_to(iq, pair.shape), 1e9)
            sel = jnp.min(cand, axis=1, keepdims=True)
            if t > 0:
                rows.append(sel)
            pair = jnp.where(iq == sel, -1e30, pair)
        idx_ref[pl.ds(jc * _G, _G)] = jnp.concatenate(rows, axis=1)
        return c

    jax.lax.fori_loop(0, _B // _G, grp, jnp.float32(0.0))

    xm = featF_ref[...] * maskF_ref[...]
    mu, var = _rowstats(xm)
    s = bng_ref[...] * jax.lax.rsqrt(var + _EPS)
    t = bnb_ref[...] - mu * s
    x0_ref[...] = (xm * s + t) * maskF_ref[...]


_EJ = _K * _P
_C = 8
_EC = _C * _EJ


def _block_body(x_ref, idx_ref, wd_ref, wb_ref, w2_ref, w3_ref,
                g1_ref, b1_ref, g2_ref, b2_ref, g3_ref, b3_ref,
                scw_ref, scg_ref, scb_ref, out_ref, u_s, v_s, e_s):
    c1 = wd_ref.shape[0]
    u_s[...] = _dot(wd_ref[...], x_ref[...])
    v_s[...] = _dot(wb_ref[...], x_ref[...])
    iq = jax.lax.broadcasted_iota(jnp.int32, (_P, _P), 0).astype(jnp.float32)

    def finalize(acc, g_ref, b_ref):
        se, sq = acc
        mean = se / _NE
        var = sq / _NE - mean * mean
        s_ = g_ref[...] * jax.lax.rsqrt(var + _EPS)
        t_ = b_ref[...] - mean * s_
        return s_, t_

    def stat_pair(e):
        return (jnp.sum(e, axis=1, keepdims=True),
                jnp.sum(e * e, axis=1, keepdims=True))

    zpad = jnp.zeros((c1, _P), jnp.bfloat16)

    def p0(j2, carry):
        se, sq = carry
        sels = []
        vs = []
        for g in range(2):
            j = j2 * 2 + g
            idxj = idx_ref[pl.ds(j, 1)].reshape(_K, _P)
            sels.append(jnp.concatenate(
                [(iq == idxj[k:k + 1, :]).astype(jnp.bfloat16)
                 for k in range(_K)],
                axis=1))
            vs.append(v_s[:, pl.ds(pl.multiple_of(j * _P, _P), _P)].astype(
                jnp.bfloat16))
        lhs = jnp.concatenate(
            [jnp.concatenate([vs[0], zpad], axis=1),
             jnp.concatenate([zpad, vs[1]], axis=1)], axis=0)
        rhs = jnp.concatenate(sels, axis=0)
        gpair = jax.lax.dot_general(
            lhs, rhs, (((1,), (0,)), ((), ())),
            preferred_element_type=jnp.float32)
        for g in range(2):
            j = j2 * 2 + g
            off = pl.ds(pl.multiple_of(j * _P, _P), _P)
            e1 = (gpair[g * c1:(g + 1) * c1, :]
                  + jnp.concatenate([u_s[:, off]] * _K, axis=1))
            e_s[:, pl.ds(pl.multiple_of(j * _EJ, _EJ), _EJ)] = e1.astype(
                jnp.bfloat16)
            ds, dq = stat_pair(e1)
            se, sq = se + ds, sq + dq
        return se, sq

    z = jnp.zeros((c1, 1), jnp.float32)
    s1, t1 = finalize(jax.lax.fori_loop(0, _B // 2, p0, (z, z)),
                      g1_ref, b1_ref)

    def p1(jc2, carry):
        se, sq = carry
        for g in range(2):
            off = pl.ds(pl.multiple_of((jc2 * 2 + g) * _EC, _EC), _EC)
            y1 = jax.nn.relu(e_s[:, off].astype(jnp.float32) * s1 + t1)
            e_s[:, off] = y1.astype(jnp.bfloat16)
            ds, dq = stat_pair(_dot(w2_ref[...], y1))
            se, sq = se + ds, sq + dq
        return se, sq

    s2, t2 = finalize(jax.lax.fori_loop(0, _B // (2 * _C), p1, (z, z)),
                      g2_ref, b2_ref)

    def p2(jc2, carry):
        se, sq = carry
        for g in range(2):
            off = pl.ds(pl.multiple_of((jc2 * 2 + g) * _EC, _EC), _EC)
            y1 = e_s[:, off].astype(jnp.float32)
            y2 = jax.nn.relu(_dot(w2_ref[...], y1) * s2 + t2)
            e3 = _dot(w3_ref[...], y2)
            e_s[:, off] = e3.astype(jnp.bfloat16)
            ds, dq = stat_pair(e3)
            se, sq = se + ds, sq + dq
        return se, sq

    s3, t3 = finalize(jax.lax.fori_loop(0, _B // (2 * _C), p2, (z, z)),
                      g3_ref, b3_ref)

    def p3(jc, c):
        off = pl.ds(pl.multiple_of(jc * _EC, _EC), _EC)
        y3 = jax.nn.relu(e_s[:, off].astype(jnp.float32) * s3 + t3)
        for j in range(_C):
            agg = y3[:, j * _EJ:j * _EJ + _P]
            for k in range(1, _K):
                base = j * _EJ + k * _P
                agg = agg + y3[:, base:base + _P]
            out_ref[:, pl.ds(pl.multiple_of((jc * _C + j) * _P, _P), _P)] = (
                agg * (1.0 / _K))
        return c

    jax.lax.fori_loop(0, _B // _C, p3, jnp.float32(0.0))

    scp = _dot(scw_ref[...], x_ref[...])
    m, v = _rowstats(scp)
    s_ = scg_ref[...] * jax.lax.rsqrt(v + _EPS)
    t_ = scb_ref[...] - m * s_
    out_ref[...] = jax.nn.relu(out_ref[...] + scp * s_ + t_)


def _tail_body(x1_ref, x2_ref, maskF_ref,
               fusew_ref, fuseg_ref, fuseb_ref,
               e1w_ref, e1b_ref, e2w_ref, e2b_ref,
               fc1w_ref, fc1b_ref, fc2w_ref, fc2b_ref,
               out_ref, ex1_ref, ex2_ref):
    ib = jax.lax.broadcasted_iota(jnp.int32, (_BP, _B), 0)
    jb = jax.lax.broadcasted_iota(jnp.int32, (_BP, _B), 1)
    emat = ((ib // _P) == jb).astype(jnp.bfloat16)
    msk = maskF_ref[...]
    cnt = jnp.maximum(_dot(msk, emat), 1.0)

    x1 = x1_ref[...]
    x2 = x2_ref[...]
    p1 = _dot(x1 * msk, emat) / cnt
    p2 = _dot(x2 * msk, emat) / cnt
    ex1_ref[...] = _dot(e1w_ref[...], p1) + e1b_ref[...]
    ex2_ref[...] = _dot(e2w_ref[...], p2) + e2b_ref[...]

    h = jnp.concatenate([x1, x2], axis=0)
    fp = _dot(fusew_ref[...], h)
    m, v = _rowstats(fp)
    s_ = fuseg_ref[...] * jax.lax.rsqrt(v + _EPS)
    t_ = fuseb_ref[...] - m * s_
    hb = jax.nn.relu(fp * s_ + t_)
    pooled = _dot(hb * (msk * msk), emat) / cnt
    z = jax.nn.relu(_dot(fc1w_ref[...], pooled) + fc1b_ref[...])
    out_ref[...] = _dot(fc2w_ref[...], z) + fc2b_ref[...]


def _col(w):
    return w.reshape(-1, 1).astype(jnp.float32)


def kernel(points, features, mask, params):
    f32 = jnp.float32
    cin = features.shape[1]
    ptsT = points.transpose(0, 2, 1)
    featF = features.transpose(1, 0, 2).reshape(cin, _BP)
    maskF = mask.transpose(1, 0, 2).reshape(1, _BP)

    idx, x0 = pl.pallas_call(
        _knn_body,
        out_shape=(jax.ShapeDtypeStruct((_B, _K, _P), f32),
                   jax.ShapeDtypeStruct((cin, _BP), f32)),
    )(points, ptsT, featF, maskF, _col(params['bn_fts_g']),
      _col(params['bn_fts_b']))

    x = x0
    block_outs = []
    for blk in params['blocks']:
        (w1, g1, b1), (w2, g2, b2), (w3, g3, b3) = blk['convs']
        c_in = x.shape[0]
        wd = w1[:, :c_in] - w1[:, c_in:]
        wb = w1[:, c_in:]
        c1, c3 = w1.shape[0], w3.shape[0]
        x = pl.pallas_call(
            _block_body,
            out_shape=jax.ShapeDtypeStruct((c3, _BP), f32),
            scratch_shapes=[pltpu.VMEM((c1, _BP), f32),
                            pltpu.VMEM((c1, _BP), f32),
                            pltpu.VMEM((c1, _NE), jnp.bfloat16)],
        )(x, idx, wd, wb, w2, w3, _col(g1), _col(b1), _col(g2), _col(b2),
          _col(g3), _col(b3), blk['sc_w'], _col(blk['sc_g']),
          _col(blk['sc_b']))
        block_outs.append(x)

    x1, x2 = block_outs
    e1, e2 = params['exits']
    ncls = params['fc2_w'].shape[0]
    out, ex1, ex2 = pl.pallas_call(
        _tail_body,
        out_shape=(jax.ShapeDtypeStruct((ncls, _B), f32),
                   jax.ShapeDtypeStruct((ncls, _B), f32),
                   jax.ShapeDtypeStruct((ncls, _B), f32)),
    )(x1, x2, maskF, params['fuse_w'], _col(params['fuse_g']),
      _col(params['fuse_b']), e1['w'], _col(e1['b']), e2['w'], _col(e2['b']),
      params['fc1_w'], _col(params['fc1_b']), params['fc2_w'],
      _col(params['fc2_b']))

    return (out.T, ex1.T, ex2.T)


# speedup vs baseline: 1.0194x; 1.0194x over previous
import jax
import jax.numpy as jnp
from jax.experimental import pallas as pl
from jax.experimental.pallas import tpu as pltpu

_B, _P, _K = 128, 128, 16
_BP = _B * _P
_NE = _BP * _K
_EPS = 1e-5


def _dot(a, b):
    return jax.lax.dot_general(
        a.astype(jnp.bfloat16), b.astype(jnp.bfloat16),
        (((1,), (0,)), ((), ())), preferred_element_type=jnp.float32)


def _rowstats(x):
    m = jnp.mean(x, axis=1, keepdims=True)
    v = jnp.mean(x * x, axis=1, keepdims=True) - m * m
    return m, v


_G = 8


def _knn_body(pts_ref, ptsT_ref, featF_ref, maskF_ref, bng_ref, bnb_ref,
              idx_ref, x0_ref):
    iq = jax.lax.broadcasted_iota(jnp.int32, (1, _P, 1), 1).astype(jnp.float32)

    def grp(jc, c):
        pj = pts_ref[pl.ds(jc * _G, _G)]
        ptj = ptsT_ref[pl.ds(jc * _G, _G)]
        px, py = pj[:, 0:1, :], pj[:, 1:2, :]
        pxc, pyc = ptj[:, :, 0:1], ptj[:, :, 1:2]
        pair = (2.0 * (pxc * px + pyc * py)
                - (pxc * pxc + pyc * pyc) - (px * px + py * py))
        rows = []
        for t in range(_K + 1):
            m = jnp.max(pair, axis=1, keepdims=True)
            cand = jnp.where(pair == m, jnp.broadcast_to(iq, pair.shape), 1e9)
            sel = jnp.min(cand, axis=1, keepdims=True)
            if t > 0:
                rows.append(sel)
            pair = jnp.where(iq == sel, -1e30, pair)
        idx_ref[pl.ds(jc * _G, _G)] = jnp.concatenate(rows, axis=1)
        return c

    jax.lax.fori_loop(0, _B // _G, grp, jnp.float32(0.0))

    xm = featF_ref[...] * maskF_ref[...]
    mu, var = _rowstats(xm)
    s = bng_ref[...] * jax.lax.rsqrt(var + _EPS)
    t = bnb_ref[...] - mu * s
    x0_ref[...] = (xm * s + t) * maskF_ref[...]


_EJ = _K * _P
_C = 8
_EC = _C * _EJ


def _block_body(x_ref, idx_ref, wd_ref, wb_ref, w2_ref, w3_ref,
                g1_ref, b1_ref, g2_ref, b2_ref, g3_ref, b3_ref,
                scw_ref, scg_ref, scb_ref, out_ref, u_s, v_s, e_s):
    c1 = wd_ref.shape[0]
    u_s[...] = _dot(wd_ref[...], x_ref[...])
    v_s[...] = _dot(wb_ref[...], x_ref[...])
    iq = jax.lax.broadcasted_iota(jnp.int32, (_P, _P), 0).astype(jnp.float32)

    def finalize(acc, g_ref, b_ref):
        se, sq = acc
        mean = se / _NE
        var = sq / _NE - mean * mean
        s_ = g_ref[...] * jax.lax.rsqrt(var + _EPS)
        t_ = b_ref[...] - mean * s_
        return s_, t_

    def stat_pair(e):
        return (jnp.sum(e, axis=1, keepdims=True),
                jnp.sum(e * e, axis=1, keepdims=True))

    def p0(j2, carry):
        se, sq = carry
        for g in range(2):
            j = j2 * 2 + g
            idxj = idx_ref[pl.ds(j, 1)].reshape(_K, _P)
            sel = jnp.concatenate(
                [(iq == idxj[k:k + 1, :]).astype(jnp.bfloat16)
                 for k in range(_K)],
                axis=1)
            off = pl.ds(pl.multiple_of(j * _P, _P), _P)
            gj = jax.lax.dot_general(
                v_s[:, off].astype(jnp.bfloat16), sel,
                (((1,), (0,)), ((), ())), preferred_element_type=jnp.float32)
            e1 = gj + jnp.concatenate([u_s[:, off]] * _K, axis=1)
            e_s[:, pl.ds(pl.multiple_of(j * _EJ, _EJ), _EJ)] = e1.astype(
                jnp.bfloat16)
            ds, dq = stat_pair(e1)
            se, sq = se + ds, sq + dq
        return se, sq

    z = jnp.zeros((c1, 1), jnp.float32)
    s1, t1 = finalize(jax.lax.fori_loop(0, _B // 2, p0, (z, z)),
                      g1_ref, b1_ref)

    def p1(jc2, carry):
        se, sq = carry
        for g in range(2):
            off = pl.ds(pl.multiple_of((jc2 * 2 + g) * _EC, _EC), _EC)
            y1 = jax.nn.relu(e_s[:, off].astype(jnp.float32) * s1 + t1)
            e_s[:, off] = y1.astype(jnp.bfloat16)
            ds, dq = stat_pair(_dot(w2_ref[...], y1))
            se, sq = se + ds, sq + dq
        return se, sq

    s2, t2 = finalize(jax.lax.fori_loop(0, _B // (2 * _C), p1, (z, z)),
                      g2_ref, b2_ref)

    def p2(jc2, carry):
        se, sq = carry
        for g in range(2):
            off = pl.ds(pl.multiple_of((jc2 * 2 + g) * _EC, _EC), _EC)
            y1 = e_s[:, off].astype(jnp.float32)
            y2 = jax.nn.relu(_dot(w2_ref[...], y1) * s2 + t2)
            e3 = _dot(w3_ref[...], y2)
            e_s[:, off] = e3.astype(jnp.bfloat16)
            ds, dq = stat_pair(e3)
            se, sq = se + ds, sq + dq
        return se, sq

    s3, t3 = finalize(jax.lax.fori_loop(0, _B // (2 * _C), p2, (z, z)),
                      g3_ref, b3_ref)

    def p3(jc, c):
        off = pl.ds(pl.multiple_of(jc * _EC, _EC), _EC)
        y3 = jax.nn.relu(e_s[:, off].astype(jnp.float32) * s3 + t3)
        for j in range(_C):
            agg = y3[:, j * _EJ:j * _EJ + _P]
            for k in range(1, _K):
                base = j * _EJ + k * _P
                agg = agg + y3[:, base:base + _P]
            out_ref[:, pl.ds(pl.multiple_of((jc * _C + j) * _P, _P), _P)] = (
                agg * (1.0 / _K))
        return c

    jax.lax.fori_loop(0, _B // _C, p3, jnp.float32(0.0))

    scp = _dot(scw_ref[...], x_ref[...])
    m, v = _rowstats(scp)
    s_ = scg_ref[...] * jax.lax.rsqrt(v + _EPS)
    t_ = scb_ref[...] - m * s_
    out_ref[...] = jax.nn.relu(out_ref[...] + scp * s_ + t_)


def _tail_body(x1_ref, x2_ref, maskF_ref,
               fusew_ref, fuseg_ref, fuseb_ref,
               e1w_ref, e1b_ref, e2w_ref, e2b_ref,
               fc1w_ref, fc1b_ref, fc2w_ref, fc2b_ref,
               out_ref, ex1_ref, ex2_ref):
    ib = jax.lax.broadcasted_iota(jnp.int32, (_BP, _B), 0)
    jb = jax.lax.broadcasted_iota(jnp.int32, (_BP, _B), 1)
    emat = ((ib // _P) == jb).astype(jnp.bfloat16)
    msk = maskF_ref[...]
    cnt = jnp.maximum(_dot(msk, emat), 1.0)

    x1 = x1_ref[...]
    x2 = x2_ref[...]
    p1 = _dot(x1 * msk, emat) / cnt
    p2 = _dot(x2 * msk, emat) / cnt
    ex1_ref[...] = _dot(e1w_ref[...], p1) + e1b_ref[...]
    ex2_ref[...] = _dot(e2w_ref[...], p2) + e2b_ref[...]

    h = jnp.concatenate([x1, x2], axis=0)
    fp = _dot(fusew_ref[...], h)
    m, v = _rowstats(fp)
    s_ = fuseg_ref[...] * jax.lax.rsqrt(v + _EPS)
    t_ = fuseb_ref[...] - m * s_
    hb = jax.nn.relu(fp * s_ + t_)
    pooled = _dot(hb * (msk * msk), emat) / cnt
    z = jax.nn.relu(_dot(fc1w_ref[...], pooled) + fc1b_ref[...])
    out_ref[...] = _dot(fc2w_ref[...], z) + fc2b_ref[...]


def _col(w):
    return w.reshape(-1, 1).astype(jnp.float32)


def kernel(points, features, mask, params):
    f32 = jnp.float32
    cin = features.shape[1]
    ptsT = points.transpose(0, 2, 1)
    featF = features.transpose(1, 0, 2).reshape(cin, _BP)
    maskF = mask.transpose(1, 0, 2).reshape(1, _BP)

    idx, x0 = pl.pallas_call(
        _knn_body,
        out_shape=(jax.ShapeDtypeStruct((_B, _K, _P), f32),
                   jax.ShapeDtypeStruct((cin, _BP), f32)),
    )(points, ptsT, featF, maskF, _col(params['bn_fts_g']),
      _col(params['bn_fts_b']))

    x = x0
    block_outs = []
    for blk in params['blocks']:
        (w1, g1, b1), (w2, g2, b2), (w3, g3, b3) = blk['convs']
        c_in = x.shape[0]
        wd = w1[:, :c_in] - w1[:, c_in:]
        wb = w1[:, c_in:]
        c1, c3 = w1.shape[0], w3.shape[0]
        x = pl.pallas_call(
            _block_body,
            out_shape=jax.ShapeDtypeStruct((c3, _BP), f32),
            scratch_shapes=[pltpu.VMEM((c1, _BP), f32),
                            pltpu.VMEM((c1, _BP), f32),
                            pltpu.VMEM((c1, _NE), jnp.bfloat16)],
        )(x, idx, wd, wb, w2, w3, _col(g1), _col(b1), _col(g2), _col(b2),
          _col(g3), _col(b3), blk['sc_w'], _col(blk['sc_g']),
          _col(blk['sc_b']))
        block_outs.append(x)

    x1, x2 = block_outs
    e1, e2 = params['exits']
    ncls = params['fc2_w'].shape[0]
    out, ex1, ex2 = pl.pallas_call(
        _tail_body,
        out_shape=(jax.ShapeDtypeStruct((ncls, _B), f32),
                   jax.ShapeDtypeStruct((ncls, _B), f32),
                   jax.ShapeDtypeStruct((ncls, _B), f32)),
    )(x1, x2, maskF, params['fuse_w'], _col(params['fuse_g']),
      _col(params['fuse_b']), e1['w'], _col(e1['b']), e2['w'], _col(e2['b']),
      params['fc1_w'], _col(params['fc1_b']), params['fc2_w'],
      _col(params['fc2_b']))

    return (out.T, ex1.T, ex2.T)
